# baseline (speedup 1.0000x reference)
import jax
import jax.numpy as jnp
from jax.experimental import pallas as pl
from jax.experimental.pallas import tpu as pltpu

B = 64
SEQ = 128
MAXA = 32
FEAT = 16
HID = 128
GNN = 128
DH = 8
NA = B * MAXA
G4 = 4 * HID
K = HID + FEAT + 1
CHUNK_T = 32
NBLK = 2
RB = NA // NBLK


def _lstm_post_kernel(x_ref, w_ref, nseg_ref, hideout_ref,
                      w1l0_ref, w1l1_ref, b10_ref, b11_ref, ss1_ref,
                      w2l2_ref, w2l3_ref, w2r_ref, b2_ref,
                      out_ref, h_ref, c_ref):
    pid = pl.program_id(0)

    @pl.when(pid == 0)
    def _init():
        h_ref[...] = jnp.zeros((NA, HID), jnp.bfloat16)
        c_ref[...] = jnp.zeros((NA, HID), jnp.bfloat16)

    w = w_ref[...]
    ones = jnp.ones((RB, 1), jnp.bfloat16)
    sig = lambda v: 0.5 + 0.5 * jnp.tanh(v)

    def step(t, carry):
        xt = x_ref[t]
        for blk in range(NBLK):
            rows = slice(blk * RB, (blk + 1) * RB)
            h = h_ref[rows, :]
            c = c_ref[rows, :]
            xh = jnp.concatenate([h, xt[rows, :], ones], axis=1)
            g = jnp.dot(xh, w, preferred_element_type=jnp.float32).astype(jnp.bfloat16)
            ig = sig(g[:, 0 * HID:1 * HID])
            fg = sig(g[:, 1 * HID:2 * HID])
            gg = jnp.tanh(g[:, 2 * HID:3 * HID])
            og = sig(g[:, 3 * HID:4 * HID])
            c2 = fg * c + ig * gg
            h2 = og * jnp.tanh(c2)
            c_ref[rows, :] = c2
            h_ref[rows, :] = h2
        return carry

    jax.lax.fori_loop(0, CHUNK_T, step, 0, unroll=16)

    @pl.when(pid == SEQ // CHUNK_T - 1)
    def _finish():
        hn = h_ref[...]
        n = nseg_ref[...]
        col = jax.lax.broadcasted_iota(jnp.int32, (B, NA), 1)
        row = jax.lax.broadcasted_iota(jnp.int32, (B, NA), 0)
        sel = (col // MAXA == row) & ((col % MAXA) < n)
        sm = jnp.where(sel, 1.0, 0.0).astype(jnp.bfloat16)
        sum_a = jnp.dot(sm, hn, preferred_element_type=jnp.float32)
        mean_a = sum_a / n.astype(jnp.float32)
        as1 = jnp.dot(mean_a, w1l0_ref[...], preferred_element_type=jnp.float32) + b10_ref[...]
        hs1 = jnp.dot(hideout_ref[...], w1l1_ref[...], preferred_element_type=jnp.float32) + b11_ref[...]
        o2 = (jnp.dot(hs1, w2l2_ref[...], preferred_element_type=jnp.float32)
              + jnp.dot(as1, w2l3_ref[...], preferred_element_type=jnp.float32)
              + jnp.dot(ss1_ref[...], w2r_ref[...], preferred_element_type=jnp.float32)
              + b2_ref[...])
        out_ref[...] = jnp.tanh(o2)


def kernel(agent_obs, hideout_obs, timestep_obs, num_agents, lstm_Wih, lstm_Whh,
           lstm_bih, lstm_bhh, sage1_Wl, sage1_Wr, sage1_b, sage2_Wl, sage2_Wr,
           sage2_b):
    del timestep_obs, sage1_Wr
    x = jnp.transpose(agent_obs.astype(jnp.bfloat16), (1, 0, 2, 3)).reshape(SEQ, NA, FEAT)
    bias_row = (lstm_bih + lstm_bhh).reshape(1, G4)
    wcat = jnp.concatenate([
        jnp.concatenate([lstm_Whh, lstm_Wih], axis=1).T,
        bias_row,
    ], axis=0)
    gate_scale = jnp.concatenate([jnp.full((HID,), 0.5), jnp.full((HID,), 0.5),
                                  jnp.ones((HID,)), jnp.full((HID,), 0.5)]).astype(jnp.float32)
    w = (wcat * gate_scale[None, :]).astype(jnp.bfloat16)
    nseg = num_agents.astype(jnp.int32).reshape(B, 1)
    ss1 = (sage1_b[2] + sage1_b[3]).reshape(1, HID)
    w2r = sage2_Wr[2] + sage2_Wr[3]
    b2 = (sage2_b[2] + sage2_b[3]).reshape(1, GNN)

    full = lambda shape: pl.BlockSpec(shape, lambda t: (0,) * len(shape))
    return pl.pallas_call(
        _lstm_post_kernel,
        grid=(SEQ // CHUNK_T,),
        in_specs=[
            pl.BlockSpec((CHUNK_T, NA, FEAT), lambda t: (t, 0, 0)),
            full((K, G4)),
            full((B, 1)),
            full((B, DH)),
            full((HID, HID)),
            full((DH, HID)),
            full((1, HID)),
            full((1, HID)),
            full((1, HID)),
            full((HID, GNN)),
            full((HID, GNN)),
            full((HID, GNN)),
            full((1, GNN)),
        ],
        out_specs=full((B, GNN)),
        out_shape=jax.ShapeDtypeStruct((B, GNN), jnp.float32),
        scratch_shapes=[
            pltpu.VMEM((NA, HID), jnp.bfloat16),
            pltpu.VMEM((NA, HID), jnp.bfloat16),
        ],
        compiler_params=pltpu.CompilerParams(
            dimension_semantics=("arbitrary",),
        ),
    )(x, w, nseg, hideout_obs.astype(jnp.float32),
      sage1_Wl[0], sage1_Wl[1], sage1_b[0].reshape(1, HID),
      sage1_b[1].reshape(1, HID), ss1,
      sage2_Wl[2], sage2_Wl[3], w2r, b2)


# speedup vs baseline: 1.1026x; 1.1026x over previous
import jax
import jax.numpy as jnp
from jax.experimental import pallas as pl
from jax.experimental.pallas import tpu as pltpu

B = 64
SEQ = 128
MAXA = 32
FEAT = 16
HID = 128
GNN = 128
DH = 8
NA = B * MAXA
G4 = 4 * HID
K = HID + FEAT + 1
CHUNK_T = 16
NBLK = 2
RB = NA // NBLK


def _lstm_post_kernel(x_ref, w_ref, nseg_ref, hideout_ref,
                      w1l0_ref, w1l1_ref, b10_ref, b11_ref, ss1_ref,
                      w2l2_ref, w2l3_ref, w2r_ref, b2_ref,
                      out_ref, h_ref, c_ref):
    pid = pl.program_id(0)

    @pl.when(pid == 0)
    def _init():
        h_ref[...] = jnp.zeros((NA, HID), jnp.bfloat16)
        c_ref[...] = jnp.zeros((NA, HID), jnp.bfloat16)

    w = w_ref[...]
    ones = jnp.ones((RB, 1), jnp.bfloat16)
    sig = lambda v: 0.5 + 0.5 * jnp.tanh(v)

    def step(t, carry):
        xt = x_ref[t].T
        for blk in range(NBLK):
            rows = slice(blk * RB, (blk + 1) * RB)
            h = h_ref[rows, :]
            c = c_ref[rows, :]
            xh = jnp.concatenate([h, xt[rows, :], ones], axis=1)
            g = jnp.dot(xh, w, preferred_element_type=jnp.float32).astype(jnp.bfloat16)
            ig = sig(g[:, 0 * HID:1 * HID])
            fg = sig(g[:, 1 * HID:2 * HID])
            gg = jnp.tanh(g[:, 2 * HID:3 * HID])
            og = sig(g[:, 3 * HID:4 * HID])
            c2 = fg * c + ig * gg
            h2 = og * jnp.tanh(c2)
            c_ref[rows, :] = c2
            h_ref[rows, :] = h2
        return carry

    jax.lax.fori_loop(0, CHUNK_T, step, 0, unroll=16)

    @pl.when(pid == SEQ // CHUNK_T - 1)
    def _finish():
        hn = h_ref[...]
        n = nseg_ref[...]
        col = jax.lax.broadcasted_iota(jnp.int32, (B, NA), 1)
        row = jax.lax.broadcasted_iota(jnp.int32, (B, NA), 0)
        sel = (col // MAXA == row) & ((col % MAXA) < n)
        sm = jnp.where(sel, 1.0, 0.0).astype(jnp.bfloat16)
        sum_a = jnp.dot(sm, hn, preferred_element_type=jnp.float32)
        mean_a = sum_a / n.astype(jnp.float32)
        as1 = jnp.dot(mean_a, w1l0_ref[...], preferred_element_type=jnp.float32) + b10_ref[...]
        hs1 = jnp.dot(hideout_ref[...], w1l1_ref[...], preferred_element_type=jnp.float32) + b11_ref[...]
        o2 = (jnp.dot(hs1, w2l2_ref[...], preferred_element_type=jnp.float32)
              + jnp.dot(as1, w2l3_ref[...], preferred_element_type=jnp.float32)
              + jnp.dot(ss1_ref[...], w2r_ref[...], preferred_element_type=jnp.float32)
              + b2_ref[...])
        out_ref[...] = jnp.tanh(o2)


def kernel(agent_obs, hideout_obs, timestep_obs, num_agents, lstm_Wih, lstm_Whh,
           lstm_bih, lstm_bhh, sage1_Wl, sage1_Wr, sage1_b, sage2_Wl, sage2_Wr,
           sage2_b):
    del timestep_obs, sage1_Wr
    x = jnp.transpose(agent_obs.astype(jnp.bfloat16), (1, 3, 0, 2)).reshape(SEQ, FEAT, NA)
    bias_row = (lstm_bih + lstm_bhh).reshape(1, G4)
    wcat = jnp.concatenate([
        jnp.concatenate([lstm_Whh, lstm_Wih], axis=1).T,
        bias_row,
    ], axis=0)
    gate_scale = jnp.concatenate([jnp.full((HID,), 0.5), jnp.full((HID,), 0.5),
                                  jnp.ones((HID,)), jnp.full((HID,), 0.5)]).astype(jnp.float32)
    w = (wcat * gate_scale[None, :]).astype(jnp.bfloat16)
    nseg = num_agents.astype(jnp.int32).reshape(B, 1)
    ss1 = (sage1_b[2] + sage1_b[3]).reshape(1, HID)
    w2r = sage2_Wr[2] + sage2_Wr[3]
    b2 = (sage2_b[2] + sage2_b[3]).reshape(1, GNN)

    full = lambda shape: pl.BlockSpec(shape, lambda t: (0,) * len(shape))
    return pl.pallas_call(
        _lstm_post_kernel,
        grid=(SEQ // CHUNK_T,),
        in_specs=[
            pl.BlockSpec((CHUNK_T, FEAT, NA), lambda t: (t, 0, 0)),
            full((K, G4)),
            full((B, 1)),
            full((B, DH)),
            full((HID, HID)),
            full((DH, HID)),
            full((1, HID)),
            full((1, HID)),
            full((1, HID)),
            full((HID, GNN)),
            full((HID, GNN)),
            full((HID, GNN)),
            full((1, GNN)),
        ],
        out_specs=full((B, GNN)),
        out_shape=jax.ShapeDtypeStruct((B, GNN), jnp.float32),
        scratch_shapes=[
            pltpu.VMEM((NA, HID), jnp.bfloat16),
            pltpu.VMEM((NA, HID), jnp.bfloat16),
        ],
        compiler_params=pltpu.CompilerParams(
            dimension_semantics=("arbitrary",),
        ),
    )(x, w, nseg, hideout_obs.astype(jnp.float32),
      sage1_Wl[0], sage1_Wl[1], sage1_b[0].reshape(1, HID),
      sage1_b[1].reshape(1, HID), ss1,
      sage2_Wl[2], sage2_Wl[3], w2r, b2)


# speedup vs baseline: 1.1220x; 1.0176x over previous
import jax
import jax.numpy as jnp
from jax.experimental import pallas as pl
from jax.experimental.pallas import tpu as pltpu

B = 64
SEQ = 128
MAXA = 32
FEAT = 16
HID = 128
GNN = 128
DH = 8
NA = B * MAXA
G4 = 4 * HID
K = HID + FEAT + 1
CHUNK_T = 16
NBLK = 2
RB = NA // NBLK


def _lstm_post_kernel(x_ref, w_ref, nseg_ref, hideout_ref,
                      w1l0_ref, w1l1_ref, b10_ref, b11_ref, ss1_ref,
                      w2l2_ref, w2l3_ref, w2r_ref, b2_ref,
                      out_ref, h_ref, c_ref):
    pid = pl.program_id(0)

    @pl.when(pid == 0)
    def _init():
        h_ref[...] = jnp.zeros((NA, HID), jnp.bfloat16)
        c_ref[...] = jnp.zeros((NA, HID), jnp.bfloat16)

    w = w_ref[...]
    ones = jnp.ones((RB, 1), jnp.bfloat16)
    sig = lambda v: 0.5 + 0.5 * jnp.tanh(v)

    def step(t, carry):
        x64 = x_ref[:, t, :]
        xt = jnp.concatenate(
            [x64[:, a * FEAT:(a + 1) * FEAT] for a in range(MAXA)],
            axis=0).astype(jnp.bfloat16)
        for blk in range(NBLK):
            rows = slice(blk * RB, (blk + 1) * RB)
            h = h_ref[rows, :]
            c = c_ref[rows, :]
            xh = jnp.concatenate([h, xt[rows, :], ones], axis=1)
            g = jnp.dot(xh, w, preferred_element_type=jnp.float32).astype(jnp.bfloat16)
            ig = sig(g[:, 0 * HID:1 * HID])
            fg = sig(g[:, 1 * HID:2 * HID])
            gg = jnp.tanh(g[:, 2 * HID:3 * HID])
            og = sig(g[:, 3 * HID:4 * HID])
            c2 = fg * c + ig * gg
            h2 = og * jnp.tanh(c2)
            c_ref[rows, :] = c2
            h_ref[rows, :] = h2
        return carry

    jax.lax.fori_loop(0, CHUNK_T, step, 0, unroll=16)

    @pl.when(pid == SEQ // CHUNK_T - 1)
    def _finish():
        hn = h_ref[...]
        n = nseg_ref[...]
        col = jax.lax.broadcasted_iota(jnp.int32, (B, NA), 1)
        row = jax.lax.broadcasted_iota(jnp.int32, (B, NA), 0)
        sel = (col % B == row) & ((col // B) < n)
        sm = jnp.where(sel, 1.0, 0.0).astype(jnp.bfloat16)
        sum_a = jnp.dot(sm, hn, preferred_element_type=jnp.float32)
        mean_a = sum_a / n.astype(jnp.float32)
        as1 = jnp.dot(mean_a, w1l0_ref[...], preferred_element_type=jnp.float32) + b10_ref[...]
        hs1 = jnp.dot(hideout_ref[...], w1l1_ref[...], preferred_element_type=jnp.float32) + b11_ref[...]
        o2 = (jnp.dot(hs1, w2l2_ref[...], preferred_element_type=jnp.float32)
              + jnp.dot(as1, w2l3_ref[...], preferred_element_type=jnp.float32)
              + jnp.dot(ss1_ref[...], w2r_ref[...], preferred_element_type=jnp.float32)
              + b2_ref[...])
        out_ref[...] = jnp.tanh(o2)


def kernel(agent_obs, hideout_obs, timestep_obs, num_agents, lstm_Wih, lstm_Whh,
           lstm_bih, lstm_bhh, sage1_Wl, sage1_Wr, sage1_b, sage2_Wl, sage2_Wr,
           sage2_b):
    del timestep_obs, sage1_Wr
    x = agent_obs.astype(jnp.float32).reshape(B, SEQ, MAXA * FEAT)
    bias_row = (lstm_bih + lstm_bhh).reshape(1, G4)
    wcat = jnp.concatenate([
        jnp.concatenate([lstm_Whh, lstm_Wih], axis=1).T,
        bias_row,
    ], axis=0)
    gate_scale = jnp.concatenate([jnp.full((HID,), 0.5), jnp.full((HID,), 0.5),
                                  jnp.ones((HID,)), jnp.full((HID,), 0.5)]).astype(jnp.float32)
    w = (wcat * gate_scale[None, :]).astype(jnp.bfloat16)
    nseg = num_agents.astype(jnp.int32).reshape(B, 1)
    ss1 = (sage1_b[2] + sage1_b[3]).reshape(1, HID)
    w2r = sage2_Wr[2] + sage2_Wr[3]
    b2 = (sage2_b[2] + sage2_b[3]).reshape(1, GNN)

    full = lambda shape: pl.BlockSpec(shape, lambda t: (0,) * len(shape))
    return pl.pallas_call(
        _lstm_post_kernel,
        grid=(SEQ // CHUNK_T,),
        in_specs=[
            pl.BlockSpec((B, CHUNK_T, MAXA * FEAT), lambda t: (0, t, 0)),
            full((K, G4)),
            full((B, 1)),
            full((B, DH)),
            full((HID, HID)),
            full((DH, HID)),
            full((1, HID)),
            full((1, HID)),
            full((1, HID)),
            full((HID, GNN)),
            full((HID, GNN)),
            full((HID, GNN)),
            full((1, GNN)),
        ],
        out_specs=full((B, GNN)),
        out_shape=jax.ShapeDtypeStruct((B, GNN), jnp.float32),
        scratch_shapes=[
            pltpu.VMEM((NA, HID), jnp.bfloat16),
            pltpu.VMEM((NA, HID), jnp.bfloat16),
        ],
        compiler_params=pltpu.CompilerParams(
            dimension_semantics=("arbitrary",),
        ),
    )(x, w, nseg, hideout_obs.astype(jnp.float32),
      sage1_Wl[0], sage1_Wl[1], sage1_b[0].reshape(1, HID),
      sage1_b[1].reshape(1, HID), ss1,
      sage2_Wl[2], sage2_Wl[3], w2r, b2)
